# baseline (speedup 1.0000x reference)
import jax
import jax.numpy as jnp
from jax.experimental import pallas as pl
from jax.experimental.pallas import tpu as pltpu

_BLOCK_ROWS = 560


def _gcn_body(adj_ref, x_ref, w_ref, out_ref, support_ref):
    @pl.when(pl.program_id(0) == 0)
    def _():
        support_ref[...] = jax.lax.dot_general(
            x_ref[...], w_ref[...],
            (((1,), (0,)), ((), ())),
            preferred_element_type=jnp.float32,
        )

    out_ref[...] = jax.lax.dot_general(
        adj_ref[...], support_ref[...],
        (((1,), (0,)), ((), ())),
        preferred_element_type=jnp.float32,
    )


def kernel(x, adj, weight):
    n_nodes, f_in = x.shape
    f_out = weight.shape[1]
    br = _BLOCK_ROWS
    grid = (n_nodes + br - 1) // br
    return pl.pallas_call(
        _gcn_body,
        grid=(grid,),
        in_specs=[
            pl.BlockSpec((br, n_nodes), lambda i: (i, 0)),
            pl.BlockSpec((n_nodes, f_in), lambda i: (0, 0)),
            pl.BlockSpec((f_in, f_out), lambda i: (0, 0)),
        ],
        out_specs=pl.BlockSpec((br, f_out), lambda i: (i, 0)),
        out_shape=jax.ShapeDtypeStruct((n_nodes, f_out), jnp.float32),
        scratch_shapes=[pltpu.VMEM((n_nodes, f_out), jnp.float32)],
        compiler_params=pltpu.CompilerParams(
            dimension_semantics=("arbitrary",),
        ),
    )(adj, x, weight)


# speedup vs baseline: 1.0196x; 1.0196x over previous
import jax
import jax.numpy as jnp
from jax.experimental import pallas as pl
from jax.experimental.pallas import tpu as pltpu

_BLOCK_ROWS = 400


def _gcn_body(adj_ref, x_ref, w_ref, out_ref, support_ref):
    @pl.when(pl.program_id(0) == 0)
    def _():
        support_ref[...] = jax.lax.dot_general(
            x_ref[...], w_ref[...],
            (((1,), (0,)), ((), ())),
            preferred_element_type=jnp.float32,
        )

    out_ref[...] = jax.lax.dot_general(
        adj_ref[...], support_ref[...],
        (((1,), (0,)), ((), ())),
        preferred_element_type=jnp.float32,
    )


def kernel(x, adj, weight):
    n_nodes, f_in = x.shape
    f_out = weight.shape[1]
    br = _BLOCK_ROWS
    grid = (n_nodes + br - 1) // br
    return pl.pallas_call(
        _gcn_body,
        grid=(grid,),
        in_specs=[
            pl.BlockSpec((br, n_nodes), lambda i: (i, 0)),
            pl.BlockSpec((n_nodes, f_in), lambda i: (0, 0)),
            pl.BlockSpec((f_in, f_out), lambda i: (0, 0)),
        ],
        out_specs=pl.BlockSpec((br, f_out), lambda i: (i, 0)),
        out_shape=jax.ShapeDtypeStruct((n_nodes, f_out), jnp.float32),
        scratch_shapes=[pltpu.VMEM((n_nodes, f_out), jnp.float32)],
        compiler_params=pltpu.CompilerParams(
            dimension_semantics=("arbitrary",),
        ),
    )(adj, x, weight)
